# baseline (speedup 1.0000x reference)
import jax
import jax.numpy as jnp
from jax.experimental import pallas as pl
from jax.experimental.pallas import tpu as pltpu

N = 10000
D_IN = 128
D_OUT = 32

ROW_BLOCK = 80


def _fused_kernel(x_ref, w_ref, adj_ref, o_ref, xw_ref):
    @pl.when(pl.program_id(0) == 0)
    def _():
        xw_ref[...] = jax.lax.dot_general(
            x_ref[...], w_ref[...],
            dimension_numbers=(((1,), (0,)), ((), ())),
            preferred_element_type=jnp.float32,
        )

    acc = jax.lax.dot_general(
        adj_ref[...], xw_ref[...],
        dimension_numbers=(((1,), (0,)), ((), ())),
        preferred_element_type=jnp.float32,
    )
    o_ref[...] = jnp.maximum(acc, 0.0)


def kernel(inputs, adj_norm, weight):
    grid = (N // ROW_BLOCK,)
    out = pl.pallas_call(
        _fused_kernel,
        grid=grid,
        in_specs=[
            pl.BlockSpec((N, D_IN), lambda i: (0, 0)),
            pl.BlockSpec((D_IN, D_OUT), lambda i: (0, 0)),
            pl.BlockSpec((ROW_BLOCK, N), lambda i: (i, 0)),
        ],
        out_specs=pl.BlockSpec((ROW_BLOCK, D_OUT), lambda i: (i, 0)),
        out_shape=jax.ShapeDtypeStruct((N, D_OUT), jnp.float32),
        scratch_shapes=[pltpu.VMEM((N, D_OUT), jnp.float32)],
        compiler_params=pltpu.CompilerParams(
            dimension_semantics=("arbitrary",),
            vmem_limit_bytes=110 * 1024 * 1024,
        ),
    )(inputs, weight, adj_norm)
    return out


# speedup vs baseline: 1.3327x; 1.3327x over previous
import jax
import jax.numpy as jnp
from jax.experimental import pallas as pl
from jax.experimental.pallas import tpu as pltpu

N = 10000
D_IN = 128
D_OUT = 32

ROW_BLOCK = 200
NBLK = N // ROW_BLOCK
NBUF = 4


def _fused_kernel(x_ref, w_ref, adj_ref, o_ref, xw_ref, buf_ref, sems):
    for s in range(NBUF):
        pltpu.make_async_copy(
            adj_ref.at[pl.ds(s * ROW_BLOCK, ROW_BLOCK), :],
            buf_ref.at[s],
            sems.at[s],
        ).start()

    xw_ref[...] = jax.lax.dot_general(
        x_ref[...], w_ref[...],
        dimension_numbers=(((1,), (0,)), ((), ())),
        preferred_element_type=jnp.float32,
    )

    for i in range(NBLK):
        s = i % NBUF
        pltpu.make_async_copy(
            adj_ref.at[pl.ds(i * ROW_BLOCK, ROW_BLOCK), :],
            buf_ref.at[s],
            sems.at[s],
        ).wait()
        acc = jax.lax.dot_general(
            buf_ref[s], xw_ref[...],
            dimension_numbers=(((1,), (0,)), ((), ())),
            preferred_element_type=jnp.float32,
        )
        o_ref[i * ROW_BLOCK:(i + 1) * ROW_BLOCK, :] = jnp.maximum(acc, 0.0)
        nxt = i + NBUF
        if nxt < NBLK:
            pltpu.make_async_copy(
                adj_ref.at[pl.ds(nxt * ROW_BLOCK, ROW_BLOCK), :],
                buf_ref.at[s],
                sems.at[s],
            ).start()


def kernel(inputs, adj_norm, weight):
    out = pl.pallas_call(
        _fused_kernel,
        in_specs=[
            pl.BlockSpec(memory_space=pltpu.VMEM),
            pl.BlockSpec(memory_space=pltpu.VMEM),
            pl.BlockSpec(memory_space=pl.ANY),
        ],
        out_specs=pl.BlockSpec(memory_space=pltpu.VMEM),
        out_shape=jax.ShapeDtypeStruct((N, D_OUT), jnp.float32),
        scratch_shapes=[
            pltpu.VMEM((N, D_OUT), jnp.float32),
            pltpu.VMEM((NBUF, ROW_BLOCK, N), jnp.float32),
            pltpu.SemaphoreType.DMA((NBUF,)),
        ],
        compiler_params=pltpu.CompilerParams(
            vmem_limit_bytes=64 * 1024 * 1024,
        ),
    )(inputs, weight, adj_norm)
    return out


# speedup vs baseline: 1.3508x; 1.0136x over previous
import jax
import jax.numpy as jnp
from jax.experimental import pallas as pl
from jax.experimental.pallas import tpu as pltpu

N = 10000
D_IN = 128
D_OUT = 32

ROW_BLOCK = 400


def _fused_kernel(x_ref, w_ref, adj_ref, o_ref, xw_ref):
    @pl.when(pl.program_id(0) == 0)
    def _():
        xw_ref[...] = jax.lax.dot_general(
            x_ref[...], w_ref[...],
            dimension_numbers=(((1,), (0,)), ((), ())),
            preferred_element_type=jnp.float32,
        )

    acc = jax.lax.dot_general(
        adj_ref[...], xw_ref[...],
        dimension_numbers=(((1,), (0,)), ((), ())),
        preferred_element_type=jnp.float32,
    )
    o_ref[...] = jnp.maximum(acc, 0.0)


def kernel(inputs, adj_norm, weight):
    grid = (N // ROW_BLOCK,)
    out = pl.pallas_call(
        _fused_kernel,
        grid=grid,
        in_specs=[
            pl.BlockSpec((N, D_IN), lambda i: (0, 0)),
            pl.BlockSpec((D_IN, D_OUT), lambda i: (0, 0)),
            pl.BlockSpec((ROW_BLOCK, N), lambda i: (i, 0)),
        ],
        out_specs=pl.BlockSpec((ROW_BLOCK, D_OUT), lambda i: (i, 0)),
        out_shape=jax.ShapeDtypeStruct((N, D_OUT), jnp.float32),
        scratch_shapes=[pltpu.VMEM((N, D_OUT), jnp.float32)],
        compiler_params=pltpu.CompilerParams(
            dimension_semantics=("arbitrary",),
        ),
    )(inputs, weight, adj_norm)
    return out
